# baseline (speedup 1.0000x reference)
import functools
import jax
import jax.numpy as jnp
import numpy as np
from jax.experimental import pallas as pl
from jax.experimental.pallas import tpu as pltpu

D_BINS = 80
TPV = (200, 704, 32)
PC_MIN = np.array([-54.0, -54.0, -5.0], dtype=np.float32)
VSIZE = np.array([0.54, 0.54, 0.25], dtype=np.float32)


def _depthnet_body(w_ref, b_ref, x_ref, weight_ref, feat_ref):
    out = jnp.dot(w_ref[...], x_ref[0], preferred_element_type=jnp.float32)
    out = out + b_ref[...]
    logits = out[:D_BINS, :]
    m = jnp.max(logits, axis=0, keepdims=True)
    e = jnp.exp(logits - m)
    s = jnp.sum(e, axis=0, keepdims=True)
    prob = e / s
    weight_ref[0] = jnp.where(prob > 1e-4, prob, 0.0)
    feat_ref[0] = out[D_BINS:, :]


def _depthnet(image_feat, W_depth, b_depth):
    B, N, C, H, W = image_feat.shape
    HW = H * W
    x = image_feat.reshape(B * N, C, HW)
    grid = (B * N,)
    weight, feat = pl.pallas_call(
        _depthnet_body,
        grid=grid,
        in_specs=[
            pl.BlockSpec((C + D_BINS, C), lambda i: (0, 0)),
            pl.BlockSpec((C + D_BINS, 1), lambda i: (0, 0)),
            pl.BlockSpec((1, C, HW), lambda i: (i, 0, 0)),
        ],
        out_specs=[
            pl.BlockSpec((1, D_BINS, HW), lambda i: (i, 0, 0)),
            pl.BlockSpec((1, C, HW), lambda i: (i, 0, 0)),
        ],
        out_shape=[
            jax.ShapeDtypeStruct((B * N, D_BINS, HW), jnp.float32),
            jax.ShapeDtypeStruct((B * N, C, HW), jnp.float32),
        ],
    )(W_depth, b_depth.reshape(-1, 1), x)
    return weight, feat


def _voxel_indices(intrinsics, extrinsics, H, W):
    ds = jnp.linspace(2.0, 50.0, D_BINS, dtype=jnp.float32)
    xs = jnp.linspace(0.0, W - 1.0, W, dtype=jnp.float32)
    ys = jnp.linspace(0.0, H - 1.0, H, dtype=jnp.float32)
    gy, gx = jnp.meshgrid(ys, xs, indexing='ij')
    gx = jnp.broadcast_to(gx[None], (D_BINS, H, W))
    gy = jnp.broadcast_to(gy[None], (D_BINS, H, W))
    gd = jnp.broadcast_to(ds[:, None, None], (D_BINS, H, W))
    uv1 = jnp.stack([gx, gy, jnp.ones_like(gx)], axis=-1)
    K_inv = jnp.linalg.inv(intrinsics)
    cam = jnp.einsum('bnij,dhwj->bndhwi', K_inv, uv1) * gd[..., None]
    cam_h = jnp.concatenate([cam, jnp.ones_like(cam[..., :1])], axis=-1)
    world = jnp.einsum('bnij,bndhwj->bndhwi', extrinsics, cam_h)[..., :3]
    vxyz = ((world - jnp.asarray(PC_MIN)) / jnp.asarray(VSIZE)).astype(jnp.int32)
    xi = jnp.clip(vxyz[..., 0], 0, TPV[1] - 1)
    yi = jnp.clip(vxyz[..., 1], 0, TPV[0] - 1)
    zi = jnp.clip(vxyz[..., 2], 0, TPV[2] - 1)
    return xi, yi, zi


def kernel(image_feat, conf_map, intrinsics, extrinsics, W_depth, b_depth):
    B, N, C, H, W = image_feat.shape
    HW = H * W
    weight, feat = _depthnet(image_feat, W_depth, b_depth)
    weight = weight.reshape(B, N, D_BINS, HW)
    feat = feat.reshape(B, N, C, HW)
    xi, yi, zi = _voxel_indices(intrinsics, extrinsics, H, W)
    xi = xi.reshape(B, N, D_BINS * HW)
    yi = yi.reshape(B, N, D_BINS * HW)
    zi = zi.reshape(B, N, D_BINS * HW)

    tpv_xy = jnp.zeros((B, TPV[0] * TPV[1], C), jnp.float32)
    tpv_xz = jnp.zeros((B, TPV[1] * TPV[2], C), jnp.float32)
    tpv_yz = jnp.zeros((B, TPV[0] * TPV[2], C), jnp.float32)
    for b in range(B):
        for n in range(N):
            f_hw = feat[b, n].T
            w_d = weight[b, n]
            weighted = (w_d[:, :, None] * f_hw[None, :, :]).reshape(-1, C)
            tpv_xy = tpv_xy.at[b, yi[b, n] * TPV[1] + xi[b, n]].add(weighted)
            tpv_xz = tpv_xz.at[b, xi[b, n] * TPV[2] + zi[b, n]].add(weighted)
            tpv_yz = tpv_yz.at[b, yi[b, n] * TPV[2] + zi[b, n]].add(weighted)
    tpv_xy = tpv_xy.reshape(B, TPV[0], TPV[1], C).transpose(0, 3, 1, 2)
    tpv_xz = tpv_xz.reshape(B, TPV[1], TPV[2], C).transpose(0, 3, 1, 2)
    tpv_yz = tpv_yz.reshape(B, TPV[0], TPV[2], C).transpose(0, 3, 1, 2)
    return tpv_xy, tpv_xz, tpv_yz
